# baseline (speedup 1.0000x reference)
import functools

import jax
import jax.numpy as jnp
from jax import lax
from jax.experimental import pallas as pl
from jax.experimental.pallas import tpu as pltpu
from jax.experimental.pallas import tpu_sc as plsc

T = 2048
D = 2048
F = 768
E = 8
K = 2

BM = 256
N = K * T
NPAD = N + E * BM
NB = NPAD // BM
NV = N // 16

NC, NS, L = 2, 16, 16
NW = NC * NS

_mesh = plsc.VectorSubcoreMesh(core_axis_name="c", subcore_axis_name="s")


def _worker_id():
    return lax.axis_index("s") * NC + lax.axis_index("c")


def _router_body(x_ref, wg_ref, ids_ref, w_ref, xpack_ref):
    xb = x_ref[...].astype(jnp.bfloat16)
    lo = jax.lax.bitcast_convert_type(xb[:, : D // 2], jnp.uint16)
    hi = jax.lax.bitcast_convert_type(xb[:, D // 2:], jnp.uint16)
    packed = lo.astype(jnp.uint32) | (hi.astype(jnp.uint32) << 16)
    xpack_ref[...] = jax.lax.bitcast_convert_type(packed, jnp.int32)

    logits = jnp.dot(x_ref[...], wg_ref[...], preferred_element_type=jnp.float32)
    m = jnp.max(logits, axis=-1, keepdims=True)
    p = jnp.exp(logits - m)
    p = p / jnp.sum(p, axis=-1, keepdims=True)
    lane = lax.broadcasted_iota(jnp.int32, p.shape, 1)
    p1 = jnp.max(p, axis=-1, keepdims=True)
    i1 = jnp.min(jnp.where(p == p1, lane, E), axis=-1, keepdims=True)
    p2m = jnp.where(lane == i1, -1.0, p)
    p2 = jnp.max(p2m, axis=-1, keepdims=True)
    i2 = jnp.min(jnp.where(p2m == p2, lane, E), axis=-1, keepdims=True)
    s = p1 + p2
    ids_ref[...] = jnp.concatenate([i1, i2], axis=1)
    w_ref[...] = jnp.concatenate([p1 / s, p2 / s], axis=1)


def _dispatch_body(ids_hbm, w_hbm, pos_hbm, tok_hbm, ws_hbm, binfo_hbm,
                   idsv, wv, rankv, posv, tokv, wsv, binfov):
    wid = _worker_id()

    @pl.when(wid == 0)
    def _():
        pltpu.sync_copy(ids_hbm, idsv)
        pltpu.sync_copy(w_hbm, wv)

        def body1(i, cnts):
            sl = pl.ds(i * L, L)
            e = idsv[sl]
            rank = jnp.zeros((L,), jnp.int32)
            new = []
            for b in range(E):
                mask = e == b
                mi = mask.astype(jnp.int32)
                c = plsc.cumsum(mi)
                rank = jnp.where(mask, c - 1 + cnts[b], rank)
                new.append(cnts[b] + jnp.sum(mi))
            rankv[sl] = rank
            return tuple(new)

        cnts = lax.fori_loop(0, NV, body1, (jnp.int32(0),) * E)

        off = [jnp.int32(0)]
        for b in range(E):
            padded = lax.shift_left(
                lax.shift_right_logical(cnts[b] + (BM - 1), 8), 8)
            off.append(off[-1] + padded)
        num_used = lax.shift_right_logical(off[E], 8)

        def bodyz(i, c):
            sl = pl.ds(i * L, L)
            j = lax.iota(jnp.int32, L) + i * L
            tokv[sl] = jnp.bitwise_and(j, T - 1)
            wsv[sl] = jnp.zeros((L,), jnp.float32)
            return c

        lax.fori_loop(0, NPAD // L, bodyz, jnp.int32(0))

        def body2(i, c):
            sl = pl.ds(i * L, L)
            e = idsv[sl]
            rank = rankv[sl]
            pos = jnp.zeros((L,), jnp.int32)
            for b in range(E):
                pos = jnp.where(e == b, rank + off[b], pos)
            j = lax.iota(jnp.int32, L) + i * L
            tok = lax.shift_right_logical(j, 1)
            kmaj = lax.shift_left(jnp.bitwise_and(j, 1), 11) + tok
            plsc.store_scatter(posv, [kmaj], pos)
            plsc.store_scatter(tokv, [pos], tok)
            plsc.store_scatter(wsv, [pos], wv[sl])
            return c

        lax.fori_loop(0, NV, body2, jnp.int32(0))

        for chunk in range(2):
            idx = lax.iota(jnp.int32, L) + chunk * L
            bs = idx * BM
            be = jnp.zeros((L,), jnp.int32)
            for b in range(E - 1):
                be = be + (bs >= off[b + 1]).astype(jnp.int32)
            be = jnp.where(idx == 24, num_used, be)
            binfov[pl.ds(chunk * L, L)] = be

        pltpu.sync_copy(posv, pos_hbm)
        pltpu.sync_copy(tokv, tok_hbm)
        pltpu.sync_copy(wsv, ws_hbm)
        pltpu.sync_copy(binfov, binfo_hbm)


_G_ROWS_W = NPAD // NW
_G_CH = 16
_G_NCH = _G_ROWS_W // _G_CH
_G_H = _G_CH // 2


def _gather_body(xb_hbm, tok_hbm, xs_hbm, idx_v, b0, b1, b2,
                 sl0, sl1, sl2, sh0, sh1, sh2, sw0, sw1, sw2):
    wid = _worker_id()
    base = wid * _G_ROWS_W
    pltpu.sync_copy(tok_hbm.at[pl.ds(base, _G_ROWS_W)], idx_v)
    bufs = (b0, b1, b2)
    sgl = (sl0, sl1, sl2)
    sgh = (sh0, sh1, sh2)
    sw = (sw0, sw1, sw2)
    glo = {}
    ghi = {}
    writes = {}
    for ci in range(_G_NCH + 2):
        if ci < _G_NCH:
            b = ci % 3
            if ci >= 3:
                writes[ci - 3].wait()
            glo[ci] = pltpu.async_copy(
                xb_hbm.at[idx_v.at[pl.ds(ci * _G_CH, _G_H)]],
                bufs[b].at[pl.ds(0, _G_H)], sgl[b])
            ghi[ci] = pltpu.async_copy(
                xb_hbm.at[idx_v.at[pl.ds(ci * _G_CH + _G_H, _G_H)]],
                bufs[b].at[pl.ds(_G_H, _G_H)], sgh[b])
        j = ci - 2
        if 0 <= j < _G_NCH:
            glo[j].wait()
            ghi[j].wait()
            writes[j] = pltpu.async_copy(
                bufs[j % 3], xs_hbm.at[pl.ds(base + j * _G_CH, _G_CH)],
                sw[j % 3])
    for j in range(_G_NCH - 3, _G_NCH):
        writes[j].wait()


def _ffn_body(binfo_ref, x_ref, ws_ref, wg_ref, wu_ref, wd_ref, out_ref):
    i = pl.program_id(0)

    @pl.when(i < binfo_ref[24])
    def _():
        xp = jax.lax.bitcast_convert_type(x_ref[...], jnp.uint32)
        lo = jax.lax.bitcast_convert_type(
            (xp & 0xFFFF).astype(jnp.uint16), jnp.bfloat16)
        hi = jax.lax.bitcast_convert_type(
            (xp >> 16).astype(jnp.uint16), jnp.bfloat16)
        x = jnp.concatenate([lo, hi], axis=1)
        g = jnp.dot(x, wg_ref[0], preferred_element_type=jnp.float32)
        u = jnp.dot(x, wu_ref[0], preferred_element_type=jnp.float32)
        h = (g * jax.nn.sigmoid(g) * u).astype(jnp.bfloat16)
        y = jnp.dot(h, wd_ref[0], preferred_element_type=jnp.float32)
        out_ref[...] = y * ws_ref[...]


_C_TOK_W = T // NW
_C_CH = 8
_C_NCH = _C_TOK_W // _C_CH


def _combine_body(y_hbm, pos_hbm, out_hbm, p0_v, p1_v,
                  r0a, r0b, r1a, r1b, sa0, sa1, sb0, sb1, sw0, sw1):
    wid = _worker_id()
    t0 = wid * _C_TOK_W
    pltpu.sync_copy(pos_hbm.at[pl.ds(t0, _C_TOK_W)], p0_v)
    pltpu.sync_copy(pos_hbm.at[pl.ds(T + t0, _C_TOK_W)], p1_v)
    r0 = (r0a, r0b)
    r1 = (r1a, r1b)
    s0 = (sa0, sa1)
    s1 = (sb0, sb1)
    sw = (sw0, sw1)
    g0 = {}
    g1 = {}
    writes = {}
    for ci in range(_C_NCH + 1):
        if ci < _C_NCH:
            b = ci % 2
            if ci >= 2:
                writes[ci - 2].wait()
            g0[ci] = pltpu.async_copy(
                y_hbm.at[p0_v.at[pl.ds(ci * _C_CH, _C_CH)]], r0[b], s0[b])
            g1[ci] = pltpu.async_copy(
                y_hbm.at[p1_v.at[pl.ds(ci * _C_CH, _C_CH)]], r1[b], s1[b])
        j = ci - 1
        if 0 <= j < _C_NCH:
            g0[j].wait()
            g1[j].wait()
            b = j % 2
            for r in range(_C_CH):
                def addb(l, cc, r=r, b=b):
                    sl = pl.ds(l * L, L)
                    r0[b][r, sl] = r0[b][r, sl] + r1[b][r, sl]
                    return cc
                lax.fori_loop(0, D // L, addb, jnp.int32(0), unroll=8)
            writes[j] = pltpu.async_copy(
                r0[b], out_hbm.at[pl.ds(t0 + j * _C_CH, _C_CH)], sw[b])
    writes[_C_NCH - 2].wait()
    writes[_C_NCH - 1].wait()


@jax.jit
def _moe(hidden_states, Wg_router, W_gate, W_up, W_down):
    f32 = jnp.float32
    i32 = jnp.int32

    ids_tk, w_tk, x_pack = pl.pallas_call(
        _router_body,
        out_shape=(jax.ShapeDtypeStruct((T, K), i32),
                   jax.ShapeDtypeStruct((T, K), f32),
                   jax.ShapeDtypeStruct((T, D // 2), i32)),
    )(hidden_states, Wg_router)
    ids_flat = ids_tk.reshape(N)
    w_flat = w_tk.reshape(N)

    dispatch = functools.partial(
        pl.kernel,
        out_type=(jax.ShapeDtypeStruct((N,), i32),
                  jax.ShapeDtypeStruct((NPAD,), i32),
                  jax.ShapeDtypeStruct((NPAD,), f32),
                  jax.ShapeDtypeStruct((2 * L,), i32)),
        mesh=_mesh,
        scratch_types=[
            pltpu.VMEM((N,), i32),
            pltpu.VMEM((N,), f32),
            pltpu.VMEM((N,), i32),
            pltpu.VMEM((N,), i32),
            pltpu.VMEM((NPAD,), i32),
            pltpu.VMEM((NPAD,), f32),
            pltpu.VMEM((2 * L,), i32),
        ],
        compiler_params=pltpu.CompilerParams(needs_layout_passes=False),
    )(_dispatch_body)
    pos, tok_sorted, w_sorted, block_info = dispatch(ids_flat, w_flat)

    gather = functools.partial(
        pl.kernel,
        out_type=jax.ShapeDtypeStruct((NPAD, D // 2), i32),
        mesh=_mesh,
        scratch_types=[
            pltpu.VMEM((_G_ROWS_W,), i32),
            pltpu.VMEM((_G_CH, D // 2), i32),
            pltpu.VMEM((_G_CH, D // 2), i32),
            pltpu.VMEM((_G_CH, D // 2), i32),
            pltpu.SemaphoreType.DMA,
            pltpu.SemaphoreType.DMA,
            pltpu.SemaphoreType.DMA,
            pltpu.SemaphoreType.DMA,
            pltpu.SemaphoreType.DMA,
            pltpu.SemaphoreType.DMA,
            pltpu.SemaphoreType.DMA,
            pltpu.SemaphoreType.DMA,
            pltpu.SemaphoreType.DMA,
        ],
    )(_gather_body)
    x_sorted = gather(x_pack, tok_sorted)

    y = pl.pallas_call(
        _ffn_body,
        grid_spec=pltpu.PrefetchScalarGridSpec(
            num_scalar_prefetch=1,
            grid=(NB,),
            in_specs=[
                pl.BlockSpec((BM, D // 2), lambda i, s: (i, 0)),
                pl.BlockSpec((BM, 1), lambda i, s: (i, 0)),
                pl.BlockSpec((1, D, F), lambda i, s: (s[i], 0, 0)),
                pl.BlockSpec((1, D, F), lambda i, s: (s[i], 0, 0)),
                pl.BlockSpec((1, F, D), lambda i, s: (s[i], 0, 0)),
            ],
            out_specs=pl.BlockSpec((BM, D), lambda i, s: (i, 0)),
        ),
        out_shape=jax.ShapeDtypeStruct((NPAD, D), f32),
        compiler_params=pltpu.CompilerParams(
            dimension_semantics=("arbitrary",),
        ),
    )(block_info, x_sorted, w_sorted.reshape(NPAD, 1),
      W_gate.astype(jnp.bfloat16), W_up.astype(jnp.bfloat16),
      W_down.astype(jnp.bfloat16))

    combine = functools.partial(
        pl.kernel,
        out_type=jax.ShapeDtypeStruct((T, D), f32),
        mesh=_mesh,
        scratch_types=[
            pltpu.VMEM((_C_TOK_W,), i32),
            pltpu.VMEM((_C_TOK_W,), i32),
            pltpu.VMEM((_C_CH, D), f32),
            pltpu.VMEM((_C_CH, D), f32),
            pltpu.VMEM((_C_CH, D), f32),
            pltpu.VMEM((_C_CH, D), f32),
            pltpu.SemaphoreType.DMA,
            pltpu.SemaphoreType.DMA,
            pltpu.SemaphoreType.DMA,
            pltpu.SemaphoreType.DMA,
            pltpu.SemaphoreType.DMA,
            pltpu.SemaphoreType.DMA,
        ],
    )(_combine_body)
    return combine(y, pos)


def kernel(hidden_states, Wg_router, W_gate, W_up, W_down):
    return _moe(hidden_states, Wg_router, W_gate, W_up, W_down)


# speedup vs baseline: 1.0628x; 1.0628x over previous
import functools

import jax
import jax.numpy as jnp
from jax import lax
from jax.experimental import pallas as pl
from jax.experimental.pallas import tpu as pltpu
from jax.experimental.pallas import tpu_sc as plsc

T = 2048
D = 2048
F = 768
E = 8
K = 2

BM = 256
N = K * T
NPAD = N + E * BM
NB = NPAD // BM
NV = N // 16

NC, NS, L = 2, 16, 16
NW = NC * NS

_mesh = plsc.VectorSubcoreMesh(core_axis_name="c", subcore_axis_name="s")


def _worker_id():
    return lax.axis_index("s") * NC + lax.axis_index("c")


def _router_body(x_ref, wg_ref, ids_ref, w_ref, xpack_ref):
    xb = x_ref[...].astype(jnp.bfloat16)
    lo = jax.lax.bitcast_convert_type(xb[:, : D // 2], jnp.uint16)
    hi = jax.lax.bitcast_convert_type(xb[:, D // 2:], jnp.uint16)
    packed = lo.astype(jnp.uint32) | (hi.astype(jnp.uint32) << 16)
    xpack_ref[...] = jax.lax.bitcast_convert_type(packed, jnp.int32)

    logits = jnp.dot(x_ref[...], wg_ref[...], preferred_element_type=jnp.float32)
    m = jnp.max(logits, axis=-1, keepdims=True)
    p = jnp.exp(logits - m)
    p = p / jnp.sum(p, axis=-1, keepdims=True)
    lane = lax.broadcasted_iota(jnp.int32, p.shape, 1)
    p1 = jnp.max(p, axis=-1, keepdims=True)
    i1 = jnp.min(jnp.where(p == p1, lane, E), axis=-1, keepdims=True)
    p2m = jnp.where(lane == i1, -1.0, p)
    p2 = jnp.max(p2m, axis=-1, keepdims=True)
    i2 = jnp.min(jnp.where(p2m == p2, lane, E), axis=-1, keepdims=True)
    s = p1 + p2
    ids_ref[...] = jnp.concatenate([i1, i2], axis=1)
    w_ref[...] = jnp.concatenate([p1 / s, p2 / s], axis=1)


_G_ROWS_W = NPAD // NW
_G_CH = 16
_G_NCH = _G_ROWS_W // _G_CH
_G_H = _G_CH // 2


def _dispatch_body(ids_hbm, w_hbm, xpack_hbm,
                   pos_hbm, ws_hbm, binfo_hbm, xs_hbm,
                   idsv, wv, rankv, posv, tokv, wsv, binfov, tok_sh,
                   idx_v, b0, b1, b2,
                   sl0, sl1, sl2, sh0, sh1, sh2, sw0, sw1, sw2):
    cid = lax.axis_index("c")
    sid = lax.axis_index("s")

    @pl.when(sid == 0)
    def _():
        pltpu.sync_copy(ids_hbm, idsv)
        pltpu.sync_copy(w_hbm, wv)

        def body1(i, cnts):
            sl = pl.ds(i * L, L)
            e = idsv[sl]
            rank = jnp.zeros((L,), jnp.int32)
            new = []
            for b in range(E):
                mask = e == b
                mi = mask.astype(jnp.int32)
                c = plsc.cumsum(mi)
                rank = jnp.where(mask, c - 1 + cnts[b], rank)
                new.append(cnts[b] + jnp.sum(mi))
            rankv[sl] = rank
            return tuple(new)

        cnts = lax.fori_loop(0, NV, body1, (jnp.int32(0),) * E)

        off = [jnp.int32(0)]
        for b in range(E):
            padded = lax.shift_left(
                lax.shift_right_logical(cnts[b] + (BM - 1), 8), 8)
            off.append(off[-1] + padded)
        num_used = lax.shift_right_logical(off[E], 8)

        def bodyz(i, c):
            sl = pl.ds(i * L, L)
            j = lax.iota(jnp.int32, L) + i * L
            tokv[sl] = jnp.bitwise_and(j, T - 1)
            wsv[sl] = jnp.zeros((L,), jnp.float32)
            return c

        lax.fori_loop(0, NPAD // L, bodyz, jnp.int32(0))

        def body2(i, c):
            sl = pl.ds(i * L, L)
            e = idsv[sl]
            rank = rankv[sl]
            pos = jnp.zeros((L,), jnp.int32)
            for b in range(E):
                pos = jnp.where(e == b, rank + off[b], pos)
            j = lax.iota(jnp.int32, L) + i * L
            tok = lax.shift_right_logical(j, 1)
            kmaj = lax.shift_left(jnp.bitwise_and(j, 1), 11) + tok
            plsc.store_scatter(posv, [kmaj], pos)
            plsc.store_scatter(tokv, [pos], tok)
            plsc.store_scatter(wsv, [pos], wv[sl])
            return c

        lax.fori_loop(0, NV, body2, jnp.int32(0))

        for chunk in range(2):
            idx = lax.iota(jnp.int32, L) + chunk * L
            bs = idx * BM
            be = jnp.zeros((L,), jnp.int32)
            for b in range(E - 1):
                be = be + (bs >= off[b + 1]).astype(jnp.int32)
            be = jnp.where(idx == 24, num_used, be)
            binfov[pl.ds(chunk * L, L)] = be

        pltpu.sync_copy(tokv, tok_sh)

        @pl.when(cid == 0)
        def _publish():
            pltpu.sync_copy(posv, pos_hbm)
            pltpu.sync_copy(wsv, ws_hbm)
            pltpu.sync_copy(binfov, binfo_hbm)

    plsc.subcore_barrier()

    base = cid * (NPAD // NC) + sid * _G_ROWS_W
    pltpu.sync_copy(tok_sh.at[pl.ds(base, _G_ROWS_W)], idx_v)
    xb_hbm = xpack_hbm
    bufs = (b0, b1, b2)
    sgl = (sl0, sl1, sl2)
    sgh = (sh0, sh1, sh2)
    sw = (sw0, sw1, sw2)
    glo = {}
    ghi = {}
    writes = {}
    for ci in range(_G_NCH + 2):
        if ci < _G_NCH:
            b = ci % 3
            if ci >= 3:
                writes[ci - 3].wait()
            glo[ci] = pltpu.async_copy(
                xb_hbm.at[idx_v.at[pl.ds(ci * _G_CH, _G_H)]],
                bufs[b].at[pl.ds(0, _G_H)], sgl[b])
            ghi[ci] = pltpu.async_copy(
                xb_hbm.at[idx_v.at[pl.ds(ci * _G_CH + _G_H, _G_H)]],
                bufs[b].at[pl.ds(_G_H, _G_H)], sgh[b])
        j = ci - 2
        if 0 <= j < _G_NCH:
            glo[j].wait()
            ghi[j].wait()
            writes[j] = pltpu.async_copy(
                bufs[j % 3], xs_hbm.at[pl.ds(base + j * _G_CH, _G_CH)],
                sw[j % 3])
    for j in range(_G_NCH - 3, _G_NCH):
        writes[j].wait()


def _ffn_body(binfo_ref, x_ref, ws_ref, wg_ref, wu_ref, wd_ref, out_ref):
    i = pl.program_id(0)

    @pl.when(i < binfo_ref[24])
    def _():
        xp = jax.lax.bitcast_convert_type(x_ref[...], jnp.uint32)
        lo = jax.lax.bitcast_convert_type(
            (xp & 0xFFFF).astype(jnp.uint16), jnp.bfloat16)
        hi = jax.lax.bitcast_convert_type(
            (xp >> 16).astype(jnp.uint16), jnp.bfloat16)
        x = jnp.concatenate([lo, hi], axis=1)
        g = jnp.dot(x, wg_ref[0], preferred_element_type=jnp.float32)
        u = jnp.dot(x, wu_ref[0], preferred_element_type=jnp.float32)
        h = (g * jax.nn.sigmoid(g) * u).astype(jnp.bfloat16)
        y = jnp.dot(h, wd_ref[0], preferred_element_type=jnp.float32)
        out_ref[...] = y * ws_ref[...]


_C_TOK_W = T // NW
_C_CH = 8
_C_NCH = _C_TOK_W // _C_CH


def _combine_body(y_hbm, pos_hbm, out_hbm, p0_v, p1_v,
                  r0a, r0b, r1a, r1b, sa0, sa1, sb0, sb1, sw0, sw1):
    wid = _worker_id()
    t0 = wid * _C_TOK_W
    pltpu.sync_copy(pos_hbm.at[pl.ds(t0, _C_TOK_W)], p0_v)
    pltpu.sync_copy(pos_hbm.at[pl.ds(T + t0, _C_TOK_W)], p1_v)
    r0 = (r0a, r0b)
    r1 = (r1a, r1b)
    s0 = (sa0, sa1)
    s1 = (sb0, sb1)
    sw = (sw0, sw1)
    g0 = {}
    g1 = {}
    writes = {}
    for ci in range(_C_NCH + 1):
        if ci < _C_NCH:
            b = ci % 2
            if ci >= 2:
                writes[ci - 2].wait()
            g0[ci] = pltpu.async_copy(
                y_hbm.at[p0_v.at[pl.ds(ci * _C_CH, _C_CH)]], r0[b], s0[b])
            g1[ci] = pltpu.async_copy(
                y_hbm.at[p1_v.at[pl.ds(ci * _C_CH, _C_CH)]], r1[b], s1[b])
        j = ci - 1
        if 0 <= j < _C_NCH:
            g0[j].wait()
            g1[j].wait()
            b = j % 2
            for r in range(_C_CH):
                def addb(l, cc, r=r, b=b):
                    sl = pl.ds(l * L, L)
                    r0[b][r, sl] = r0[b][r, sl] + r1[b][r, sl]
                    return cc
                lax.fori_loop(0, D // L, addb, jnp.int32(0))
            writes[j] = pltpu.async_copy(
                r0[b], out_hbm.at[pl.ds(t0 + j * _C_CH, _C_CH)], sw[b])
    writes[_C_NCH - 2].wait()
    writes[_C_NCH - 1].wait()


@jax.jit
def _moe(hidden_states, Wg_router, W_gate, W_up, W_down):
    f32 = jnp.float32
    i32 = jnp.int32

    ids_tk, w_tk, x_pack = pl.pallas_call(
        _router_body,
        out_shape=(jax.ShapeDtypeStruct((T, K), i32),
                   jax.ShapeDtypeStruct((T, K), f32),
                   jax.ShapeDtypeStruct((T, D // 2), i32)),
    )(hidden_states, Wg_router)
    ids_flat = ids_tk.reshape(N)
    w_flat = w_tk.reshape(N)

    dispatch = functools.partial(
        pl.kernel,
        out_type=(jax.ShapeDtypeStruct((N,), i32),
                  jax.ShapeDtypeStruct((NPAD,), f32),
                  jax.ShapeDtypeStruct((2 * L,), i32),
                  jax.ShapeDtypeStruct((NPAD, D // 2), i32)),
        mesh=_mesh,
        scratch_types=[
            pltpu.VMEM((N,), i32),
            pltpu.VMEM((N,), f32),
            pltpu.VMEM((N,), i32),
            pltpu.VMEM((N,), i32),
            pltpu.VMEM((NPAD,), i32),
            pltpu.VMEM((NPAD,), f32),
            pltpu.VMEM((2 * L,), i32),
            pltpu.VMEM_SHARED((NPAD,), i32),
            pltpu.VMEM((_G_ROWS_W,), i32),
            pltpu.VMEM((_G_CH, D // 2), i32),
            pltpu.VMEM((_G_CH, D // 2), i32),
            pltpu.VMEM((_G_CH, D // 2), i32),
            pltpu.SemaphoreType.DMA,
            pltpu.SemaphoreType.DMA,
            pltpu.SemaphoreType.DMA,
            pltpu.SemaphoreType.DMA,
            pltpu.SemaphoreType.DMA,
            pltpu.SemaphoreType.DMA,
            pltpu.SemaphoreType.DMA,
            pltpu.SemaphoreType.DMA,
            pltpu.SemaphoreType.DMA,
        ],
        compiler_params=pltpu.CompilerParams(needs_layout_passes=False),
    )(_dispatch_body)
    pos, w_sorted, block_info, x_sorted = dispatch(ids_flat, w_flat, x_pack)

    y = pl.pallas_call(
        _ffn_body,
        grid_spec=pltpu.PrefetchScalarGridSpec(
            num_scalar_prefetch=1,
            grid=(NB,),
            in_specs=[
                pl.BlockSpec((BM, D // 2), lambda i, s: (i, 0)),
                pl.BlockSpec((BM, 1), lambda i, s: (i, 0)),
                pl.BlockSpec((1, D, F), lambda i, s: (s[i], 0, 0)),
                pl.BlockSpec((1, D, F), lambda i, s: (s[i], 0, 0)),
                pl.BlockSpec((1, F, D), lambda i, s: (s[i], 0, 0)),
            ],
            out_specs=pl.BlockSpec((BM, D), lambda i, s: (i, 0)),
        ),
        out_shape=jax.ShapeDtypeStruct((NPAD, D), f32),
        compiler_params=pltpu.CompilerParams(
            dimension_semantics=("arbitrary",),
        ),
    )(block_info, x_sorted, w_sorted.reshape(NPAD, 1),
      W_gate.astype(jnp.bfloat16), W_up.astype(jnp.bfloat16),
      W_down.astype(jnp.bfloat16))

    combine = functools.partial(
        pl.kernel,
        out_type=jax.ShapeDtypeStruct((T, D), f32),
        mesh=_mesh,
        scratch_types=[
            pltpu.VMEM((_C_TOK_W,), i32),
            pltpu.VMEM((_C_TOK_W,), i32),
            pltpu.VMEM((_C_CH, D), f32),
            pltpu.VMEM((_C_CH, D), f32),
            pltpu.VMEM((_C_CH, D), f32),
            pltpu.VMEM((_C_CH, D), f32),
            pltpu.SemaphoreType.DMA,
            pltpu.SemaphoreType.DMA,
            pltpu.SemaphoreType.DMA,
            pltpu.SemaphoreType.DMA,
            pltpu.SemaphoreType.DMA,
            pltpu.SemaphoreType.DMA,
        ],
    )(_combine_body)
    return combine(y, pos)


def kernel(hidden_states, Wg_router, W_gate, W_up, W_down):
    return _moe(hidden_states, Wg_router, W_gate, W_up, W_down)
